# Initial kernel scaffold; baseline (speedup 1.0000x reference)
import jax
import jax.numpy as jnp
from jax.experimental import pallas as pl


def kernel(pos, table):
    raise NotImplementedError("write your pallas kernel here")


# speedup vs baseline: 1.9824x; 1.9824x over previous
import functools

import jax
import jax.numpy as jnp
from jax import lax
from jax.experimental import pallas as pl
from jax.experimental.pallas import tpu as pltpu
from jax.experimental.pallas import tpu_sc as plsc

EMB = 1024
CH = 32


def _make_gather(B):
    info = plsc.get_sparse_core_info()
    NC, NS = info.num_cores, info.num_subcores
    NW = NC * NS
    assert B % NW == 0
    b_per_w = B // NW
    assert b_per_w % CH == 0
    nch = b_per_w // CH

    mesh = plsc.VectorSubcoreMesh(core_axis_name="c", subcore_axis_name="s")

    @functools.partial(
        pl.kernel,
        mesh=mesh,
        out_type=jax.ShapeDtypeStruct((B, EMB), jnp.float32),
        scratch_types=[
            pltpu.VMEM((b_per_w,), jnp.int32),
            pltpu.VMEM((CH, EMB), jnp.float32),
            pltpu.SemaphoreType.DMA,
        ],
    )
    def gather_kernel(table_hbm, idx_hbm, out_hbm, idx_v, rows_v, sem):
        wid = lax.axis_index("s") * NC + lax.axis_index("c")
        base = wid * b_per_w
        pltpu.sync_copy(idx_hbm.at[pl.ds(base, b_per_w)], idx_v)

        def chunk(c, carry):
            off = c * CH
            pltpu.async_copy(
                table_hbm.at[idx_v.at[pl.ds(off, CH)]], rows_v, sem
            ).wait()
            pltpu.sync_copy(rows_v, out_hbm.at[pl.ds(base + off, CH)])
            return carry

        lax.fori_loop(0, nch, chunk, 0)

    return gather_kernel


def kernel(pos, table):
    b, s = pos.shape
    flat = pos.reshape(b * s)
    out = _make_gather(b * s)(table, flat)
    return out.reshape(b, s, EMB)


# speedup vs baseline: 2.3061x; 1.1633x over previous
import functools

import jax
import jax.numpy as jnp
from jax import lax
from jax.experimental import pallas as pl
from jax.experimental.pallas import tpu as pltpu
from jax.experimental.pallas import tpu_sc as plsc

EMB = 1024
CH = 32


def _make_gather(B):
    info = plsc.get_sparse_core_info()
    NC, NS = info.num_cores, info.num_subcores
    NW = NC * NS
    assert B % NW == 0
    b_per_w = B // NW
    assert b_per_w % (2 * CH) == 0
    nch = b_per_w // CH

    mesh = plsc.VectorSubcoreMesh(core_axis_name="c", subcore_axis_name="s")

    @functools.partial(
        pl.kernel,
        mesh=mesh,
        out_type=jax.ShapeDtypeStruct((B, EMB), jnp.float32),
        scratch_types=[
            pltpu.VMEM((b_per_w,), jnp.int32),
            pltpu.VMEM((2, CH, EMB), jnp.float32),
            pltpu.SemaphoreType.DMA,
            pltpu.SemaphoreType.DMA,
        ],
    )
    def gather_kernel(table_hbm, idx_hbm, out_hbm, idx_v, rows_v, gsem, osem):
        wid = lax.axis_index("s") * NC + lax.axis_index("c")
        base = wid * b_per_w
        pltpu.sync_copy(idx_hbm.at[pl.ds(base, b_per_w)], idx_v)

        def gather_start(i, p):
            pltpu.async_copy(
                table_hbm.at[idx_v.at[pl.ds(i * CH, CH)]], rows_v.at[p], gsem
            )

        def gather_wait(p):
            pltpu.make_async_copy(
                table_hbm.at[idx_v.at[pl.ds(0, CH)]], rows_v.at[p], gsem
            ).wait()

        def out_start(i, p):
            pltpu.async_copy(
                rows_v.at[p], out_hbm.at[pl.ds(base + i * CH, CH)], osem
            )

        def out_wait(p):
            pltpu.make_async_copy(
                rows_v.at[p], out_hbm.at[pl.ds(base, CH)], osem
            ).wait()

        gather_start(0, 0)

        def step(c, carry):
            for p in range(2):
                i = 2 * c + p
                gather_wait(p)

                @pl.when(i > 0)
                def _():
                    out_wait(1 - p)

                @pl.when(i + 1 < nch)
                def _():
                    gather_start(i + 1, 1 - p)

                out_start(i, p)
            return carry

        lax.fori_loop(0, nch // 2, step, 0)
        out_wait((nch - 1) % 2)

    return gather_kernel


def kernel(pos, table):
    b, s = pos.shape
    flat = pos.reshape(b * s)
    out = _make_gather(b * s)(table, flat)
    return out.reshape(b, s, EMB)


# speedup vs baseline: 2.3976x; 1.0396x over previous
import functools

import jax
import jax.numpy as jnp
from jax import lax
from jax.experimental import pallas as pl
from jax.experimental.pallas import tpu as pltpu
from jax.experimental.pallas import tpu_sc as plsc

EMB = 1024
CH = 16
NBUF = 4


def _make_gather(B):
    info = plsc.get_sparse_core_info()
    NC, NS = info.num_cores, info.num_subcores
    NW = NC * NS
    assert B % NW == 0
    b_per_w = B // NW
    assert b_per_w % (NBUF * CH) == 0
    nch = b_per_w // CH

    mesh = plsc.VectorSubcoreMesh(core_axis_name="c", subcore_axis_name="s")

    @functools.partial(
        pl.kernel,
        mesh=mesh,
        out_type=jax.ShapeDtypeStruct((B, EMB), jnp.float32),
        scratch_types=[
            pltpu.VMEM((b_per_w,), jnp.int32),
            pltpu.VMEM((NBUF, CH, EMB), jnp.float32),
        ]
        + [pltpu.SemaphoreType.DMA] * (2 * NBUF),
    )
    def gather_kernel(table_hbm, idx_hbm, out_hbm, idx_v, rows_v, *sems):
        gsems, osems = sems[:NBUF], sems[NBUF:]
        wid = lax.axis_index("s") * NC + lax.axis_index("c")
        base = wid * b_per_w
        pltpu.sync_copy(idx_hbm.at[pl.ds(base, b_per_w)], idx_v)

        def gather_start(i, p):
            pltpu.async_copy(
                table_hbm.at[idx_v.at[pl.ds(i * CH, CH)]], rows_v.at[p], gsems[p]
            )

        def gather_wait(p):
            pltpu.make_async_copy(
                table_hbm.at[idx_v.at[pl.ds(0, CH)]], rows_v.at[p], gsems[p]
            ).wait()

        def out_start(i, p):
            pltpu.async_copy(
                rows_v.at[p], out_hbm.at[pl.ds(base + i * CH, CH)], osems[p]
            )

        def out_wait(p):
            pltpu.make_async_copy(
                rows_v.at[p], out_hbm.at[pl.ds(base, CH)], osems[p]
            ).wait()

        for p in range(NBUF - 1):
            gather_start(p, p)

        def step(c, carry):
            for p in range(NBUF):
                i = NBUF * c + p
                gather_wait(p)
                prev = (p - 1) % NBUF

                @pl.when(i > 0)
                def _():
                    out_wait(prev)

                @pl.when(i + NBUF - 1 < nch)
                def _():
                    gather_start(i + NBUF - 1, prev)

                out_start(i, p)
            return carry

        lax.fori_loop(0, nch // NBUF, step, 0)
        out_wait((nch - 1) % NBUF)

    return gather_kernel


def kernel(pos, table):
    b, s = pos.shape
    flat = pos.reshape(b * s)
    out = _make_gather(b * s)(table, flat)
    return out.reshape(b, s, EMB)
